# Initial kernel scaffold; baseline (speedup 1.0000x reference)
import jax
import jax.numpy as jnp
from jax.experimental import pallas as pl


def kernel(features, output, weight, bias):
    raise NotImplementedError("write your pallas kernel here")


# speedup vs baseline: 4.8910x; 4.8910x over previous
import jax
import jax.numpy as jnp
from jax.experimental import pallas as pl
from jax.experimental.pallas import tpu as pltpu

B, C, D = 32, 100000, 64
K = D // 2
P = 0.5
BC = 8192
NB = -(-C // BC)


def _dropout_const():
    dk = jax.random.fold_in(jax.random.key(42), 7)
    keep = jax.random.bernoulli(dk, 1.0 - P, (B, K))
    return keep.astype(jnp.float32) / (1.0 - P)


def _body(feat_ref, w_ref, bias_ref, out_ref, drop_ref, res_ref, m_ref, s_ref):
    i = pl.program_id(0)

    @pl.when(i == 0)
    def _init():
        m_ref[...] = jnp.full((B, 1), -jnp.inf, jnp.float32)
        s_ref[...] = jnp.zeros((B, 1), jnp.float32)

    bi = (i + 1) % NB
    blk = out_ref[...]
    lane_w = jax.lax.broadcasted_iota(jnp.int32, (B, BC), 1)
    valid = (bi * BC + lane_w) < C
    blk_m = jnp.where(valid, blk, -jnp.inf)
    m_old = m_ref[...]
    m_new = jnp.maximum(m_old, jnp.max(blk_m, axis=1, keepdims=True))
    e = jnp.where(valid, jnp.exp(blk - m_new), 0.0)
    s_ref[...] = (s_ref[...] * jnp.exp(m_old - m_new)
                  + jnp.sum(e, axis=1, keepdims=True))
    m_ref[...] = m_new

    feat = feat_ref[...]
    w_blk = w_ref[...]
    full = jax.lax.dot_general(feat, w_blk, (((1,), (1,)), ((), ())),
                               preferred_element_type=jnp.float32)
    plain = full + bias_ref[...]

    @pl.when(i != NB - 1)
    def _store_plain():
        res_ref[...] = plain

    @pl.when(i == NB - 1)
    def _fixup():
        m = m_ref[...]
        s = s_ref[...]
        out0 = blk[:, 0:1]
        h = jnp.exp(out0 - m) / s

        eye = (jax.lax.broadcasted_iota(jnp.int32, (B, B), 0)
               == jax.lax.broadcasted_iota(jnp.int32, (B, B), 1)).astype(jnp.float32)
        hrow = jnp.sum(eye * h, axis=0, keepdims=True)
        bcol = jax.lax.broadcasted_iota(jnp.int32, (B, B), 0)
        brow = jax.lax.broadcasted_iota(jnp.int32, (B, B), 1)
        gt = (hrow > h).astype(jnp.int32)
        eq_lo = ((hrow == h) & (brow < bcol)).astype(jnp.int32)
        rank = jnp.sum(gt + eq_lo, axis=1, keepdims=True)

        jrow = jax.lax.broadcasted_iota(jnp.int32, (B, B), 1)
        rank_oh = (rank == jrow).astype(jnp.float32)
        drow = jax.lax.dot_general(rank_oh, drop_ref[...],
                                   (((1,), (0,)), ((), ())),
                                   preferred_element_type=jnp.float32)

        w32 = w_blk[0:B, :]
        lane = jax.lax.broadcasted_iota(jnp.int32, (B, D), 1)
        erank = jnp.zeros((B, D), jnp.int32)
        for dp in range(D):
            colv = w32[:, dp:dp + 1]
            erank = erank + (colv > w32).astype(jnp.int32)
            erank = erank + ((colv == w32) & (lane > dp)).astype(jnp.int32)

        maskval = jnp.ones((B, D), jnp.float32)
        for j in range(K):
            maskval = jnp.where(erank == j, drow[:, j:j + 1], maskval)

        wm = w32 * maskval
        fix = jax.lax.dot_general(feat, wm, (((1,), (1,)), ((), ())),
                                  preferred_element_type=jnp.float32)
        sel = (jax.lax.broadcasted_iota(jnp.int32, (B, BC), 0)
               == jax.lax.broadcasted_iota(jnp.int32, (B, BC), 1)).astype(jnp.float32)
        fix_wide = jax.lax.dot_general(fix, sel, (((1,), (0,)), ((), ())),
                                       preferred_element_type=jnp.float32)
        lane_wide = jax.lax.broadcasted_iota(jnp.int32, (B, BC), 1)
        res_ref[...] = jnp.where(lane_wide < B, fix_wide, plain)


def kernel(features, output, weight, bias):
    dropped = _dropout_const()
    bias2d = bias.reshape(1, C)
    shift = lambda i: (i + 1) % NB
    return pl.pallas_call(
        _body,
        grid=(NB,),
        in_specs=[
            pl.BlockSpec((B, D), lambda i: (0, 0)),
            pl.BlockSpec((BC, D), lambda i: (shift(i), 0)),
            pl.BlockSpec((1, BC), lambda i: (0, shift(i))),
            pl.BlockSpec((B, BC), lambda i: (0, shift(i))),
            pl.BlockSpec((B, K), lambda i: (0, 0)),
        ],
        out_specs=pl.BlockSpec((B, BC), lambda i: (0, shift(i))),
        out_shape=jax.ShapeDtypeStruct((B, C), jnp.float32),
        scratch_shapes=[
            pltpu.VMEM((B, 1), jnp.float32),
            pltpu.VMEM((B, 1), jnp.float32),
        ],
    )(features, weight, bias2d, output, dropped)
